# baseline (speedup 1.0000x reference)
import functools

import jax
import jax.numpy as jnp
from jax import lax
from jax.experimental import pallas as pl
from jax.experimental.pallas import tpu as pltpu
from jax.experimental.pallas import tpu_sc as plsc

_N = 10000
_E = 320000
_H = 128
_EB = 128
_NB = _E // _EB
_NW = 32
_NU = 40
_WR = 80
_NHI = 80


def _node_prep_body(nm, nf, wro, bro, wmsgh, nr_out, nmsg_out):
    x = jnp.dot(nm[...], wro[:128, :], preferred_element_type=jnp.float32)
    x = x + jnp.dot(nf[...], wro[128:, :], preferred_element_type=jnp.float32)
    nr = jnp.maximum(x + bro[...], 0.0)
    nr_out[...] = nr
    nmsg_out[...] = jnp.dot(nr, wmsgh[...], preferred_element_type=jnp.float32)


def _edge_prep_body(ef, te, wmsge, wmsgt, bmsg, ep_out):
    x = jnp.dot(ef[...], wmsge[...], preferred_element_type=jnp.float32)
    x = x + jnp.dot(te[...], wmsgt[...], preferred_element_type=jnp.float32)
    ep_out[...] = x + bmsg[...]


def _cnt_hist_body(seg, out):
    s = seg[...][:, 0]
    hi = jnp.equal(
        lax.broadcasted_iota(jnp.int32, (_NHI, s.shape[0]), 0),
        (s // 128)[None, :]).astype(jnp.float32)
    lo = jnp.equal(
        lax.broadcasted_iota(jnp.int32, (s.shape[0], 128), 1),
        (s % 128)[:, None]).astype(jnp.float32)
    c = jnp.dot(hi, lo, preferred_element_type=jnp.float32)

    @pl.when(pl.program_id(0) == 0)
    def _():
        out[...] = jnp.zeros_like(out)

    out[...] += c


def _finish_body(p, c, nr, wagg, bagg, wupd, bupd, ww, bw, out):
    msum = p[0] + p[1]
    cnt = c[...]
    mean = msum / jnp.maximum(cnt, 1.0)
    nrv = nr[...]
    a = jnp.dot(nrv, wagg[:128, :], preferred_element_type=jnp.float32)
    a = a + jnp.dot(mean, wagg[128:, :], preferred_element_type=jnp.float32)
    agg = jnp.maximum(a + bagg[...], 0.0)
    u = jnp.dot(agg, wupd[:128, :], preferred_element_type=jnp.float32)
    u = u + jnp.dot(nrv, wupd[128:, :], preferred_element_type=jnp.float32)
    upd = jnp.maximum(u + bupd[...], 0.0)
    out[...] = jnp.tanh(
        jnp.dot(upd, ww[...], preferred_element_type=jnp.float32) + bw[...])


def _sc_body(nmsg_hbm, ep_hbm, src_hbm, seg_hbm, out_msg,
             acc_sh, gbufs, ebuf, srcvs, segis, gsems, esem):
    ci = lax.axis_index("c")
    si = lax.axis_index("s")
    wid = si * 2 + ci

    zero16 = jnp.zeros((16,), jnp.float32)

    @pl.loop(0, _WR)
    def _(i):
        for j in range(8):
            ebuf[i, pl.ds(16 * j, 16)] = zero16

    @pl.loop(si, _N // _WR, step=16)
    def _(ch):
        pltpu.sync_copy(ebuf.at[pl.ds(0, _WR)], acc_sh.at[pl.ds(ch * _WR, _WR)])

    plsc.subcore_barrier()

    def stage_idx_gather(k, b):
        @pl.when(b < _NB)
        def _():
            pltpu.sync_copy(src_hbm.at[b], srcvs[k])
            pltpu.async_copy(nmsg_hbm.at[srcvs[k]], gbufs[k], gsems[k])

    def stage_seg(k, b):
        @pl.when(b < _NB)
        def _():
            pltpu.sync_copy(seg_hbm.at[b], segis[k])

    def stage_ep(b):
        @pl.when(b < _NB)
        def _():
            pltpu.async_copy(ep_hbm.at[pl.ds(b * _EB, _EB)], ebuf, esem)

    def process(k, b, bnext):
        @pl.when(b < _NB)
        def _():
            pltpu.make_async_copy(
                nmsg_hbm.at[srcvs[k]], gbufs[k], gsems[k]).wait()
            pltpu.make_async_copy(
                ep_hbm.at[pl.ds(b * _EB, _EB)], ebuf, esem).wait()

            @pl.loop(0, _EB, unroll=4)
            def _(i):
                for j in range(8):
                    sl = pl.ds(16 * j, 16)
                    ebuf[i, sl] = jnp.maximum(
                        gbufs[k][i, sl] + ebuf[i, sl], 0.0)

            stage_idx_gather(k, bnext)
            pltpu.sync_copy(ebuf, acc_sh.at[segis[k]], add=True)
            stage_seg(k, bnext)
            stage_ep(b + _NW)

    stage_idx_gather(0, wid)
    stage_seg(0, wid)
    stage_idx_gather(1, wid + _NW)
    stage_seg(1, wid + _NW)
    stage_ep(wid)

    @pl.loop(0, _NU)
    def _(u):
        b0 = wid + u * 2 * _NW
        process(0, b0, b0 + 2 * _NW)
        process(1, b0 + _NW, b0 + 3 * _NW)

    plsc.subcore_barrier()

    @pl.loop(si, _N // _WR, step=16)
    def _(ch):
        sl = pl.ds(ch * _WR, _WR)
        pltpu.sync_copy(acc_sh.at[sl], ebuf.at[pl.ds(0, _WR)])
        pltpu.sync_copy(ebuf.at[pl.ds(0, _WR)], out_msg.at[ci].at[sl])


def _segment_mean_sc(node_msg, edge_part, source_ids, segment_ids):
    sidx2d = source_ids.reshape(_NB, _EB).astype(jnp.int32)
    seg2d = segment_ids.reshape(_NB, _EB).astype(jnp.int32)
    mesh = plsc.VectorSubcoreMesh(core_axis_name="c", subcore_axis_name="s")
    f = pl.kernel(
        _sc_body,
        out_type=jax.ShapeDtypeStruct((2, _N, _H), jnp.float32),
        mesh=mesh,
        scratch_types=[
            pltpu.VMEM_SHARED((_N, _H), jnp.float32),
            [pltpu.VMEM((_EB, _H), jnp.float32) for _ in (0, 1)],
            pltpu.VMEM((_EB, _H), jnp.float32),
            [pltpu.VMEM((_EB,), jnp.int32) for _ in (0, 1)],
            [pltpu.VMEM((_EB,), jnp.int32) for _ in (0, 1)],
            [pltpu.SemaphoreType.DMA for _ in (0, 1)],
            pltpu.SemaphoreType.DMA,
        ],
    )
    return f(node_msg, edge_part, sidx2d, seg2d)


@jax.jit
def _run(node_memory, node_features, edge_features, time_encoding,
         source_ids, segment_ids,
         W_ro, b_ro, W_msg, b_msg, W_agg, b_agg, W_upd, b_upd, W_w, b_w):
    b_ro2 = b_ro.reshape(1, _H)
    b_msg2 = b_msg.reshape(1, _H)
    b_agg2 = b_agg.reshape(1, _H)
    b_upd2 = b_upd.reshape(1, _H)
    b_w2 = b_w.reshape(1, -1)

    nblk = 2000
    node_readout, node_msg = pl.pallas_call(
        _node_prep_body,
        grid=(_N // nblk,),
        in_specs=[
            pl.BlockSpec((nblk, _H), lambda i: (i, 0)),
            pl.BlockSpec((nblk, _H), lambda i: (i, 0)),
            pl.BlockSpec((256, _H), lambda i: (0, 0)),
            pl.BlockSpec((1, _H), lambda i: (0, 0)),
            pl.BlockSpec((_H, _H), lambda i: (0, 0)),
        ],
        out_specs=[
            pl.BlockSpec((nblk, _H), lambda i: (i, 0)),
            pl.BlockSpec((nblk, _H), lambda i: (i, 0)),
        ],
        out_shape=[
            jax.ShapeDtypeStruct((_N, _H), jnp.float32),
            jax.ShapeDtypeStruct((_N, _H), jnp.float32),
        ],
    )(node_memory, node_features, W_ro, b_ro2, W_msg[:_H])

    eblk = 6400
    edge_part = pl.pallas_call(
        _edge_prep_body,
        grid=(_E // eblk,),
        in_specs=[
            pl.BlockSpec((eblk, 16), lambda i: (i, 0)),
            pl.BlockSpec((eblk, 16), lambda i: (i, 0)),
            pl.BlockSpec((16, _H), lambda i: (0, 0)),
            pl.BlockSpec((16, _H), lambda i: (0, 0)),
            pl.BlockSpec((1, _H), lambda i: (0, 0)),
        ],
        out_specs=pl.BlockSpec((eblk, _H), lambda i: (i, 0)),
        out_shape=jax.ShapeDtypeStruct((_E, _H), jnp.float32),
    )(edge_features, time_encoding, W_msg[_H:_H + 16], W_msg[_H + 16:], b_msg2)

    msg_p = _segment_mean_sc(node_msg, edge_part, source_ids, segment_ids)

    hblk = 3200
    cnt2d = pl.pallas_call(
        _cnt_hist_body,
        grid=(_E // hblk,),
        in_specs=[pl.BlockSpec((hblk, 1), lambda i: (i, 0))],
        out_specs=pl.BlockSpec((_NHI, 128), lambda i: (0, 0)),
        out_shape=jax.ShapeDtypeStruct((_NHI, 128), jnp.float32),
    )(segment_ids.astype(jnp.int32).reshape(_E, 1))
    cnt = cnt2d.reshape(-1)[:_N].reshape(_N, 1)

    fblk = 2000
    out = pl.pallas_call(
        _finish_body,
        grid=(_N // fblk,),
        in_specs=[
            pl.BlockSpec((2, fblk, _H), lambda i: (0, i, 0)),
            pl.BlockSpec((fblk, 1), lambda i: (i, 0)),
            pl.BlockSpec((fblk, _H), lambda i: (i, 0)),
            pl.BlockSpec((256, _H), lambda i: (0, 0)),
            pl.BlockSpec((1, _H), lambda i: (0, 0)),
            pl.BlockSpec((256, _H), lambda i: (0, 0)),
            pl.BlockSpec((1, _H), lambda i: (0, 0)),
            pl.BlockSpec((_H, 128), lambda i: (0, 0)),
            pl.BlockSpec((1, 128), lambda i: (0, 0)),
        ],
        out_specs=pl.BlockSpec((fblk, 128), lambda i: (i, 0)),
        out_shape=jax.ShapeDtypeStruct((_N, 128), jnp.float32),
    )(msg_p, cnt, node_readout, W_agg, b_agg2, W_upd, b_upd2, W_w, b_w2)
    return out


def kernel(node_memory, node_features, edge_features, time_encoding,
           source_ids, segment_ids, node_ids,
           W_ro, b_ro, W_msg, b_msg, W_agg, b_agg, W_upd, b_upd, W_w, b_w):
    del node_ids
    return _run(node_memory, node_features, edge_features, time_encoding,
                source_ids, segment_ids,
                W_ro, b_ro, W_msg, b_msg, W_agg, b_agg, W_upd, b_upd, W_w, b_w)


# speedup vs baseline: 1.2454x; 1.2454x over previous
import functools

import jax
import jax.numpy as jnp
from jax import lax
from jax.experimental import pallas as pl
from jax.experimental.pallas import tpu as pltpu
from jax.experimental.pallas import tpu_sc as plsc

_N = 10000
_E = 320000
_H = 128
_EB = 128
_NB = _E // _EB
_NW = 32
_NU = 40
_WR = 80
_NHI = 80


def _node_prep_body(nm, nf, wro, bro, wmsgh, nr_out, nmsg_out):
    x = jnp.dot(nm[...], wro[:128, :], preferred_element_type=jnp.float32)
    x = x + jnp.dot(nf[...], wro[128:, :], preferred_element_type=jnp.float32)
    nr = jnp.maximum(x + bro[...], 0.0)
    nr_out[...] = nr
    nmsg_out[...] = jnp.dot(nr, wmsgh[...], preferred_element_type=jnp.float32)


def _edge_prep_body(ef, te, wmsge, wmsgt, bmsg, ep_out):
    x = jnp.dot(ef[...], wmsge[...], preferred_element_type=jnp.float32)
    x = x + jnp.dot(te[...], wmsgt[...], preferred_element_type=jnp.float32)
    ep_out[...] = x + bmsg[...]


def _cnt_hist_body(seg, out):
    s = seg[...][:, 0]
    hi = jnp.equal(
        lax.broadcasted_iota(jnp.int32, (_NHI, s.shape[0]), 0),
        (s // 128)[None, :]).astype(jnp.float32)
    lo = jnp.equal(
        lax.broadcasted_iota(jnp.int32, (s.shape[0], 128), 1),
        (s % 128)[:, None]).astype(jnp.float32)
    c = jnp.dot(hi, lo, preferred_element_type=jnp.float32)

    @pl.when(pl.program_id(0) == 0)
    def _():
        out[...] = jnp.zeros_like(out)

    out[...] += c


def _finish_body(p, c, nr, wagg, bagg, wupd, bupd, ww, bw, out):
    msum = p[0] + p[1]
    cnt = c[...]
    mean = msum / jnp.maximum(cnt, 1.0)
    nrv = nr[...]
    a = jnp.dot(nrv, wagg[:128, :], preferred_element_type=jnp.float32)
    a = a + jnp.dot(mean, wagg[128:, :], preferred_element_type=jnp.float32)
    agg = jnp.maximum(a + bagg[...], 0.0)
    u = jnp.dot(agg, wupd[:128, :], preferred_element_type=jnp.float32)
    u = u + jnp.dot(nrv, wupd[128:, :], preferred_element_type=jnp.float32)
    upd = jnp.maximum(u + bupd[...], 0.0)
    out[...] = jnp.tanh(
        jnp.dot(upd, ww[...], preferred_element_type=jnp.float32) + bw[...])


def _sc_body(nmsg_hbm, ep_hbm, src_hbm, seg_hbm, out_msg,
             acc_sh, gbufs, ebuf, srcvs, segis):
    ci = lax.axis_index("c")
    si = lax.axis_index("s")
    wid = si * 2 + ci

    zero16 = jnp.zeros((16,), jnp.float32)

    @pl.loop(0, _WR)
    def _(i):
        for j in range(8):
            ebuf[i, pl.ds(16 * j, 16)] = zero16

    @pl.loop(si, _N // _WR, step=16)
    def _(ch):
        pltpu.sync_copy(ebuf.at[pl.ds(0, _WR)], acc_sh.at[pl.ds(ch * _WR, _WR)])

    plsc.subcore_barrier()

    gbuf = gbufs[0]

    @pl.loop(wid, _NB, step=_NW)
    def _(b):
        pltpu.sync_copy(src_hbm.at[b], srcvs[0])
        pltpu.sync_copy(seg_hbm.at[b], segis[0])
        pltpu.sync_copy(ep_hbm.at[pl.ds(b * _EB, _EB)], ebuf)
        pltpu.sync_copy(nmsg_hbm.at[srcvs[0]], gbuf)

        @pl.loop(0, _EB)
        def _(i):
            for j in range(8):
                sl = pl.ds(16 * j, 16)
                gbuf[i, sl] = jnp.maximum(gbuf[i, sl] + ebuf[i, sl], 0.0)

        pltpu.sync_copy(gbuf, acc_sh.at[segis[0]], add=True)

    plsc.subcore_barrier()

    @pl.loop(si, _N // _WR, step=16)
    def _(ch):
        sl = pl.ds(ch * _WR, _WR)
        pltpu.sync_copy(acc_sh.at[sl], ebuf.at[pl.ds(0, _WR)])
        pltpu.sync_copy(ebuf.at[pl.ds(0, _WR)], out_msg.at[ci].at[sl])


def _segment_mean_sc(node_msg, edge_part, source_ids, segment_ids):
    sidx2d = source_ids.reshape(_NB, _EB).astype(jnp.int32)
    seg2d = segment_ids.reshape(_NB, _EB).astype(jnp.int32)
    mesh = plsc.VectorSubcoreMesh(core_axis_name="c", subcore_axis_name="s")
    f = pl.kernel(
        _sc_body,
        out_type=jax.ShapeDtypeStruct((2, _N, _H), jnp.float32),
        mesh=mesh,
        scratch_types=[
            pltpu.VMEM_SHARED((_N, _H), jnp.float32),
            [pltpu.VMEM((_EB, _H), jnp.float32)],
            pltpu.VMEM((_EB, _H), jnp.float32),
            [pltpu.VMEM((_EB,), jnp.int32)],
            [pltpu.VMEM((_EB,), jnp.int32)],
        ],
    )
    return f(node_msg, edge_part, sidx2d, seg2d)


@jax.jit
def _run(node_memory, node_features, edge_features, time_encoding,
         source_ids, segment_ids,
         W_ro, b_ro, W_msg, b_msg, W_agg, b_agg, W_upd, b_upd, W_w, b_w):
    b_ro2 = b_ro.reshape(1, _H)
    b_msg2 = b_msg.reshape(1, _H)
    b_agg2 = b_agg.reshape(1, _H)
    b_upd2 = b_upd.reshape(1, _H)
    b_w2 = b_w.reshape(1, -1)

    nblk = 2000
    node_readout, node_msg = pl.pallas_call(
        _node_prep_body,
        grid=(_N // nblk,),
        in_specs=[
            pl.BlockSpec((nblk, _H), lambda i: (i, 0)),
            pl.BlockSpec((nblk, _H), lambda i: (i, 0)),
            pl.BlockSpec((256, _H), lambda i: (0, 0)),
            pl.BlockSpec((1, _H), lambda i: (0, 0)),
            pl.BlockSpec((_H, _H), lambda i: (0, 0)),
        ],
        out_specs=[
            pl.BlockSpec((nblk, _H), lambda i: (i, 0)),
            pl.BlockSpec((nblk, _H), lambda i: (i, 0)),
        ],
        out_shape=[
            jax.ShapeDtypeStruct((_N, _H), jnp.float32),
            jax.ShapeDtypeStruct((_N, _H), jnp.float32),
        ],
    )(node_memory, node_features, W_ro, b_ro2, W_msg[:_H])

    eblk = 6400
    edge_part = pl.pallas_call(
        _edge_prep_body,
        grid=(_E // eblk,),
        in_specs=[
            pl.BlockSpec((eblk, 16), lambda i: (i, 0)),
            pl.BlockSpec((eblk, 16), lambda i: (i, 0)),
            pl.BlockSpec((16, _H), lambda i: (0, 0)),
            pl.BlockSpec((16, _H), lambda i: (0, 0)),
            pl.BlockSpec((1, _H), lambda i: (0, 0)),
        ],
        out_specs=pl.BlockSpec((eblk, _H), lambda i: (i, 0)),
        out_shape=jax.ShapeDtypeStruct((_E, _H), jnp.float32),
    )(edge_features, time_encoding, W_msg[_H:_H + 16], W_msg[_H + 16:], b_msg2)

    msg_p = _segment_mean_sc(node_msg, edge_part, source_ids, segment_ids)

    hblk = 3200
    cnt2d = pl.pallas_call(
        _cnt_hist_body,
        grid=(_E // hblk,),
        in_specs=[pl.BlockSpec((hblk, 1), lambda i: (i, 0))],
        out_specs=pl.BlockSpec((_NHI, 128), lambda i: (0, 0)),
        out_shape=jax.ShapeDtypeStruct((_NHI, 128), jnp.float32),
    )(segment_ids.astype(jnp.int32).reshape(_E, 1))
    cnt = cnt2d.reshape(-1)[:_N].reshape(_N, 1)

    fblk = 2000
    out = pl.pallas_call(
        _finish_body,
        grid=(_N // fblk,),
        in_specs=[
            pl.BlockSpec((2, fblk, _H), lambda i: (0, i, 0)),
            pl.BlockSpec((fblk, 1), lambda i: (i, 0)),
            pl.BlockSpec((fblk, _H), lambda i: (i, 0)),
            pl.BlockSpec((256, _H), lambda i: (0, 0)),
            pl.BlockSpec((1, _H), lambda i: (0, 0)),
            pl.BlockSpec((256, _H), lambda i: (0, 0)),
            pl.BlockSpec((1, _H), lambda i: (0, 0)),
            pl.BlockSpec((_H, 128), lambda i: (0, 0)),
            pl.BlockSpec((1, 128), lambda i: (0, 0)),
        ],
        out_specs=pl.BlockSpec((fblk, 128), lambda i: (i, 0)),
        out_shape=jax.ShapeDtypeStruct((_N, 128), jnp.float32),
    )(msg_p, cnt, node_readout, W_agg, b_agg2, W_upd, b_upd2, W_w, b_w2)
    return out


def kernel(node_memory, node_features, edge_features, time_encoding,
           source_ids, segment_ids, node_ids,
           W_ro, b_ro, W_msg, b_msg, W_agg, b_agg, W_upd, b_upd, W_w, b_w):
    del node_ids
    return _run(node_memory, node_features, edge_features, time_encoding,
                source_ids, segment_ids,
                W_ro, b_ro, W_msg, b_msg, W_agg, b_agg, W_upd, b_upd, W_w, b_w)


# speedup vs baseline: 1.4004x; 1.1244x over previous
import functools

import jax
import jax.numpy as jnp
from jax import lax
from jax.experimental import pallas as pl
from jax.experimental.pallas import tpu as pltpu
from jax.experimental.pallas import tpu_sc as plsc

_N = 10000
_E = 320000
_H = 128
_EB = 128
_NB = _E // _EB
_NW = 32
_NU = 40
_WR = 80
_NHI = 80


def _node_prep_body(nm, nf, wro, bro, wmsgh, nr_out, nmsg_out):
    x = jnp.dot(nm[...], wro[:128, :], preferred_element_type=jnp.float32)
    x = x + jnp.dot(nf[...], wro[128:, :], preferred_element_type=jnp.float32)
    nr = jnp.maximum(x + bro[...], 0.0)
    nr_out[...] = nr
    nmsg_out[...] = jnp.dot(nr, wmsgh[...], preferred_element_type=jnp.float32)


def _edge_prep_body(ef, te, wmsge, wmsgt, bmsg, ep_out):
    x = jnp.dot(ef[...], wmsge[...], preferred_element_type=jnp.float32)
    x = x + jnp.dot(te[...], wmsgt[...], preferred_element_type=jnp.float32)
    ep_out[...] = x + bmsg[...]


def _cnt_hist_body(seg, out):
    s = seg[...][:, 0]
    hi = jnp.equal(
        lax.broadcasted_iota(jnp.int32, (_NHI, s.shape[0]), 0),
        (s // 128)[None, :]).astype(jnp.float32)
    lo = jnp.equal(
        lax.broadcasted_iota(jnp.int32, (s.shape[0], 128), 1),
        (s % 128)[:, None]).astype(jnp.float32)
    c = jnp.dot(hi, lo, preferred_element_type=jnp.float32)

    @pl.when(pl.program_id(0) == 0)
    def _():
        out[...] = jnp.zeros_like(out)

    out[...] += c


def _finish_body(p, c, nr, wagg, bagg, wupd, bupd, ww, bw, out):
    msum = p[0] + p[1]
    cnt = c[...]
    mean = msum / jnp.maximum(cnt, 1.0)
    nrv = nr[...]
    a = jnp.dot(nrv, wagg[:128, :], preferred_element_type=jnp.float32)
    a = a + jnp.dot(mean, wagg[128:, :], preferred_element_type=jnp.float32)
    agg = jnp.maximum(a + bagg[...], 0.0)
    u = jnp.dot(agg, wupd[:128, :], preferred_element_type=jnp.float32)
    u = u + jnp.dot(nrv, wupd[128:, :], preferred_element_type=jnp.float32)
    upd = jnp.maximum(u + bupd[...], 0.0)
    out[...] = jnp.tanh(
        jnp.dot(upd, ww[...], preferred_element_type=jnp.float32) + bw[...])


def _sc_body(nmsg_hbm, ep_hbm, src_hbm, seg_hbm, out_msg,
             acc_sh, gbufs, ebuf, srcvs, segis, esem):
    ci = lax.axis_index("c")
    si = lax.axis_index("s")
    wid = si * 2 + ci

    zero16 = jnp.zeros((16,), jnp.float32)

    @pl.loop(0, _WR)
    def _(i):
        for j in range(8):
            ebuf[i, pl.ds(16 * j, 16)] = zero16

    @pl.loop(si, _N // _WR, step=16)
    def _(ch):
        pltpu.sync_copy(ebuf.at[pl.ds(0, _WR)], acc_sh.at[pl.ds(ch * _WR, _WR)])

    plsc.subcore_barrier()

    gbuf = gbufs[0]

    @pl.loop(wid, _NB, step=_NW)
    def _(b):
        pltpu.async_copy(ep_hbm.at[pl.ds(b * _EB, _EB)], ebuf, esem)
        pltpu.sync_copy(src_hbm.at[b], srcvs[0])
        pltpu.sync_copy(seg_hbm.at[b], segis[0])
        pltpu.sync_copy(nmsg_hbm.at[srcvs[0]], gbuf)
        pltpu.make_async_copy(ep_hbm.at[pl.ds(b * _EB, _EB)], ebuf, esem).wait()

        @pl.loop(0, _EB)
        def _(i):
            for j in range(8):
                sl = pl.ds(16 * j, 16)
                gbuf[i, sl] = jnp.maximum(gbuf[i, sl] + ebuf[i, sl], 0.0)

        pltpu.sync_copy(gbuf, acc_sh.at[segis[0]], add=True)

    plsc.subcore_barrier()

    @pl.loop(si, _N // _WR, step=16)
    def _(ch):
        sl = pl.ds(ch * _WR, _WR)
        pltpu.sync_copy(acc_sh.at[sl], ebuf.at[pl.ds(0, _WR)])
        pltpu.sync_copy(ebuf.at[pl.ds(0, _WR)], out_msg.at[ci].at[sl])


def _segment_mean_sc(node_msg, edge_part, source_ids, segment_ids):
    sidx2d = source_ids.reshape(_NB, _EB).astype(jnp.int32)
    seg2d = segment_ids.reshape(_NB, _EB).astype(jnp.int32)
    mesh = plsc.VectorSubcoreMesh(core_axis_name="c", subcore_axis_name="s")
    f = pl.kernel(
        _sc_body,
        out_type=jax.ShapeDtypeStruct((2, _N, _H), jnp.float32),
        mesh=mesh,
        scratch_types=[
            pltpu.VMEM_SHARED((_N, _H), jnp.float32),
            [pltpu.VMEM((_EB, _H), jnp.float32)],
            pltpu.VMEM((_EB, _H), jnp.float32),
            [pltpu.VMEM((_EB,), jnp.int32)],
            [pltpu.VMEM((_EB,), jnp.int32)],
            pltpu.SemaphoreType.DMA,
        ],
    )
    return f(node_msg, edge_part, sidx2d, seg2d)


@jax.jit
def _run(node_memory, node_features, edge_features, time_encoding,
         source_ids, segment_ids,
         W_ro, b_ro, W_msg, b_msg, W_agg, b_agg, W_upd, b_upd, W_w, b_w):
    b_ro2 = b_ro.reshape(1, _H)
    b_msg2 = b_msg.reshape(1, _H)
    b_agg2 = b_agg.reshape(1, _H)
    b_upd2 = b_upd.reshape(1, _H)
    b_w2 = b_w.reshape(1, -1)

    nblk = 2000
    node_readout, node_msg = pl.pallas_call(
        _node_prep_body,
        grid=(_N // nblk,),
        in_specs=[
            pl.BlockSpec((nblk, _H), lambda i: (i, 0)),
            pl.BlockSpec((nblk, _H), lambda i: (i, 0)),
            pl.BlockSpec((256, _H), lambda i: (0, 0)),
            pl.BlockSpec((1, _H), lambda i: (0, 0)),
            pl.BlockSpec((_H, _H), lambda i: (0, 0)),
        ],
        out_specs=[
            pl.BlockSpec((nblk, _H), lambda i: (i, 0)),
            pl.BlockSpec((nblk, _H), lambda i: (i, 0)),
        ],
        out_shape=[
            jax.ShapeDtypeStruct((_N, _H), jnp.float32),
            jax.ShapeDtypeStruct((_N, _H), jnp.float32),
        ],
    )(node_memory, node_features, W_ro, b_ro2, W_msg[:_H])

    eblk = 6400
    edge_part = pl.pallas_call(
        _edge_prep_body,
        grid=(_E // eblk,),
        in_specs=[
            pl.BlockSpec((eblk, 16), lambda i: (i, 0)),
            pl.BlockSpec((eblk, 16), lambda i: (i, 0)),
            pl.BlockSpec((16, _H), lambda i: (0, 0)),
            pl.BlockSpec((16, _H), lambda i: (0, 0)),
            pl.BlockSpec((1, _H), lambda i: (0, 0)),
        ],
        out_specs=pl.BlockSpec((eblk, _H), lambda i: (i, 0)),
        out_shape=jax.ShapeDtypeStruct((_E, _H), jnp.float32),
    )(edge_features, time_encoding, W_msg[_H:_H + 16], W_msg[_H + 16:], b_msg2)

    msg_p = _segment_mean_sc(node_msg, edge_part, source_ids, segment_ids)

    hblk = 3200
    cnt2d = pl.pallas_call(
        _cnt_hist_body,
        grid=(_E // hblk,),
        in_specs=[pl.BlockSpec((hblk, 1), lambda i: (i, 0))],
        out_specs=pl.BlockSpec((_NHI, 128), lambda i: (0, 0)),
        out_shape=jax.ShapeDtypeStruct((_NHI, 128), jnp.float32),
    )(segment_ids.astype(jnp.int32).reshape(_E, 1))
    cnt = cnt2d.reshape(-1)[:_N].reshape(_N, 1)

    fblk = 2000
    out = pl.pallas_call(
        _finish_body,
        grid=(_N // fblk,),
        in_specs=[
            pl.BlockSpec((2, fblk, _H), lambda i: (0, i, 0)),
            pl.BlockSpec((fblk, 1), lambda i: (i, 0)),
            pl.BlockSpec((fblk, _H), lambda i: (i, 0)),
            pl.BlockSpec((256, _H), lambda i: (0, 0)),
            pl.BlockSpec((1, _H), lambda i: (0, 0)),
            pl.BlockSpec((256, _H), lambda i: (0, 0)),
            pl.BlockSpec((1, _H), lambda i: (0, 0)),
            pl.BlockSpec((_H, 128), lambda i: (0, 0)),
            pl.BlockSpec((1, 128), lambda i: (0, 0)),
        ],
        out_specs=pl.BlockSpec((fblk, 128), lambda i: (i, 0)),
        out_shape=jax.ShapeDtypeStruct((_N, 128), jnp.float32),
    )(msg_p, cnt, node_readout, W_agg, b_agg2, W_upd, b_upd2, W_w, b_w2)
    return out


def kernel(node_memory, node_features, edge_features, time_encoding,
           source_ids, segment_ids, node_ids,
           W_ro, b_ro, W_msg, b_msg, W_agg, b_agg, W_upd, b_upd, W_w, b_w):
    del node_ids
    return _run(node_memory, node_features, edge_features, time_encoding,
                source_ids, segment_ids,
                W_ro, b_ro, W_msg, b_msg, W_agg, b_agg, W_upd, b_upd, W_w, b_w)
